# baseline (speedup 1.0000x reference)
import functools

import jax
import jax.numpy as jnp
from jax import lax
from jax.experimental import pallas as pl
from jax.experimental.pallas import tpu as pltpu
from jax.experimental.pallas import tpu_sc as plsc

_N = 10000
_D = 128
_E = 320000
_NC = 2
_NS = 16
_NW = _NC * _NS
_L = 16
_CHUNK = 128
_CHUNKS_PER_TILE = -(-_E // (_NW * _CHUNK))
_CHUNKS_PER_TILE += _CHUNKS_PER_TILE % 2
_EDGES_PER_TILE = _CHUNKS_PER_TILE * _CHUNK
_E_PAD = _NW * _EDGES_PER_TILE
_AGG_ROWS = 10112
_ZBLOCKS = _AGG_ROWS // _CHUNK


def _unpack_chunk(pack_v, j, sidx, didx):
    for l in range(_CHUNK // _L):
        w = pack_v[j, pl.ds(l * _L, _L)]
        sidx[0, pl.ds(l * _L, _L)] = lax.shift_right_logical(w, 16)
        didx[0, pl.ds(l * _L, _L)] = lax.bitwise_and(w, 0xFFFF)


def _sc_body(x_hbm, pack_hbm, zeros_hbm, out_hbm,
             pack_v, sidx_a, sidx_b, didx_a, didx_b, rows_a, rows_b,
             agg_sh, sem_ga, sem_gb, sem_s):
    c = lax.axis_index("c")
    s = lax.axis_index("s")
    wid = c * _NS + s
    n = _CHUNKS_PER_TILE

    pltpu.sync_copy(zeros_hbm, rows_a)
    for k in range(-(-_ZBLOCKS // _NS)):
        blk = s + k * _NS

        @pl.when(blk < _ZBLOCKS)
        def _():
            pltpu.sync_copy(rows_a, agg_sh.at[pl.ds(blk * _CHUNK, _CHUNK)])

    plsc.subcore_barrier()

    pltpu.sync_copy(pack_hbm.at[wid], pack_v)

    bufs = ((rows_a, sidx_a, didx_a, sem_ga), (rows_b, sidx_b, didx_b, sem_gb))

    _unpack_chunk(pack_v, 0, sidx_a, didx_a)
    _unpack_chunk(pack_v, 1, sidx_b, didx_b)
    pltpu.async_copy(x_hbm.at[sidx_a.at[0]], rows_a, sem_ga)
    pltpu.async_copy(x_hbm.at[sidx_b.at[0]], rows_b, sem_gb)

    def pair_body(i, carry):
        j0 = 2 * i
        for bi, (rows, sidx, didx, sem_g) in enumerate(bufs):
            j = j0 + bi
            pltpu.make_async_copy(x_hbm.at[sidx.at[0]], rows, sem_g).wait()
            pltpu.async_copy(rows, agg_sh.at[didx.at[0]], sem_s, add=True).wait()

            @pl.when(j + 2 < n)
            def _():
                _unpack_chunk(pack_v, j + 2, sidx, didx)
                pltpu.async_copy(x_hbm.at[sidx.at[0]], rows, sem_g)

        return carry

    lax.fori_loop(0, n // 2, pair_body, 0, unroll=False)

    plsc.subcore_barrier()

    nfull = _N // _CHUNK
    tail = _N - nfull * _CHUNK
    for k in range(-(-(nfull + 1) // _NS)):
        blk = s + k * _NS
        r0 = pl.multiple_of(blk * _CHUNK, _CHUNK)
        o0 = pl.multiple_of(c * _N + r0, 16)

        @pl.when(blk < nfull)
        def _():
            pltpu.sync_copy(agg_sh.at[pl.ds(r0, _CHUNK)], rows_a)
            pltpu.sync_copy(rows_a, out_hbm.at[pl.ds(o0, _CHUNK)])

        @pl.when(blk == nfull)
        def _():
            pltpu.sync_copy(agg_sh.at[pl.ds(r0, tail)], rows_a.at[pl.ds(0, tail)])
            pltpu.sync_copy(rows_a.at[pl.ds(0, tail)],
                            out_hbm.at[pl.ds(o0, tail)])


@functools.cache
def _sc_segment_sum():
    mesh = plsc.VectorSubcoreMesh(
        core_axis_name="c", subcore_axis_name="s", num_cores=_NC, num_subcores=_NS
    )
    return pl.kernel(
        _sc_body,
        out_type=jax.ShapeDtypeStruct((_NC * _N, _D), jnp.float32),
        mesh=mesh,
        scratch_types=[
            pltpu.VMEM((_CHUNKS_PER_TILE, _CHUNK), jnp.int32),
            pltpu.VMEM((1, _CHUNK), jnp.int32),
            pltpu.VMEM((1, _CHUNK), jnp.int32),
            pltpu.VMEM((1, _CHUNK), jnp.int32),
            pltpu.VMEM((1, _CHUNK), jnp.int32),
            pltpu.VMEM((_CHUNK, _D), jnp.float32),
            pltpu.VMEM((_CHUNK, _D), jnp.float32),
            pltpu.VMEM_SHARED((_AGG_ROWS, _D), jnp.float32),
            pltpu.SemaphoreType.DMA,
            pltpu.SemaphoreType.DMA,
            pltpu.SemaphoreType.DMA,
        ],
    )


_TC_ROWS = 1000


def _tc_body(x_ref, agg_ref, wm_ref, ws_ref, b_ref, o_ref):
    agg = agg_ref[0] + agg_ref[1]
    acc = jnp.dot(x_ref[...], ws_ref[...], preferred_element_type=jnp.float32)
    acc = acc + jnp.dot(agg, wm_ref[...], preferred_element_type=jnp.float32)
    o_ref[...] = jnp.maximum(acc + b_ref[...], 0.0)


@jax.jit
def _tc_combine(x, agg2, W_msg, W_self, b2):
    return pl.pallas_call(
        _tc_body,
        grid=(_N // _TC_ROWS,),
        in_specs=[
            pl.BlockSpec((_TC_ROWS, _D), lambda i: (i, 0)),
            pl.BlockSpec((_NC, _TC_ROWS, _D), lambda i: (0, i, 0)),
            pl.BlockSpec((_D, _D), lambda i: (0, 0)),
            pl.BlockSpec((_D, _D), lambda i: (0, 0)),
            pl.BlockSpec((1, _D), lambda i: (0, 0)),
        ],
        out_specs=pl.BlockSpec((_TC_ROWS, _D), lambda i: (i, 0)),
        out_shape=jax.ShapeDtypeStruct((_N, _D), jnp.float32),
    )(x, agg2, W_msg, W_self, b2)


def kernel(x, edge_index, W_msg, W_self, b):
    src = edge_index[0].astype(jnp.int32)
    dst = edge_index[1].astype(jnp.int32)
    packed = jnp.concatenate([
        (src << 16) | dst,
        jnp.full((_E_PAD - _E,), _N, jnp.int32),
    ]).reshape(_NW, _CHUNKS_PER_TILE, _CHUNK)
    zeros_blk = jnp.zeros((_CHUNK, _D), jnp.float32)
    agg2 = _sc_segment_sum()(x, packed, zeros_blk).reshape(_NC, _N, _D)
    return _tc_combine(x, agg2, W_msg, W_self, b.reshape(1, _D))


# speedup vs baseline: 1.0011x; 1.0011x over previous
import functools

import jax
import jax.numpy as jnp
from jax import lax
from jax.experimental import pallas as pl
from jax.experimental.pallas import tpu as pltpu
from jax.experimental.pallas import tpu_sc as plsc

_N = 10000
_D = 128
_E = 320000
_NC = 2
_NS = 16
_NW = _NC * _NS
_L = 16
_CHUNK = 128
_CHUNKS_PER_TILE = -(-_E // (_NW * _CHUNK))
_CHUNKS_PER_TILE += _CHUNKS_PER_TILE % 2
_EDGES_PER_TILE = _CHUNKS_PER_TILE * _CHUNK
_E_PAD = _NW * _EDGES_PER_TILE
_AGG_ROWS = 10112
_ZBLOCKS = _AGG_ROWS // _CHUNK


def _unpack_chunk(pack_v, j, sidx, didx):
    for l in range(_CHUNK // _L):
        w = pack_v[j, pl.ds(l * _L, _L)]
        sidx[0, pl.ds(l * _L, _L)] = lax.shift_right_logical(w, 16)
        didx[0, pl.ds(l * _L, _L)] = lax.bitwise_and(w, 0xFFFF)


def _sc_body(x_hbm, pack_hbm, zeros_hbm, out_hbm,
             pack_v, sidx_a, sidx_b, didx_a, didx_b, rows_a, rows_b,
             agg_sh, sem_ga, sem_gb, sem_s):
    c = lax.axis_index("c")
    s = lax.axis_index("s")
    wid = c * _NS + s
    n = _CHUNKS_PER_TILE

    with jax.named_scope("sc_zero"):
        pltpu.sync_copy(zeros_hbm, rows_a)
        for k in range(-(-_ZBLOCKS // _NS)):
            blk = s + k * _NS

            @pl.when(blk < _ZBLOCKS)
            def _():
                pltpu.sync_copy(rows_a, agg_sh.at[pl.ds(blk * _CHUNK, _CHUNK)])

        plsc.subcore_barrier()

    pltpu.sync_copy(pack_hbm.at[wid], pack_v)

    bufs = ((rows_a, sidx_a, didx_a, sem_ga), (rows_b, sidx_b, didx_b, sem_gb))

    edge_scope = jax.named_scope("sc_edges")
    edge_scope.__enter__()
    _unpack_chunk(pack_v, 0, sidx_a, didx_a)
    _unpack_chunk(pack_v, 1, sidx_b, didx_b)
    pltpu.async_copy(x_hbm.at[sidx_a.at[0]], rows_a, sem_ga)
    pltpu.async_copy(x_hbm.at[sidx_b.at[0]], rows_b, sem_gb)

    def pair_body(i, carry):
        j0 = 2 * i
        for bi, (rows, sidx, didx, sem_g) in enumerate(bufs):
            j = j0 + bi
            pltpu.make_async_copy(x_hbm.at[sidx.at[0]], rows, sem_g).wait()
            pltpu.async_copy(rows, agg_sh.at[didx.at[0]], sem_s, add=True).wait()

            @pl.when(j + 2 < n)
            def _():
                _unpack_chunk(pack_v, j + 2, sidx, didx)
                pltpu.async_copy(x_hbm.at[sidx.at[0]], rows, sem_g)

        return carry

    lax.fori_loop(0, n // 2, pair_body, 0, unroll=False)
    edge_scope.__exit__(None, None, None)

    plsc.subcore_barrier()

    nfull = _N // _CHUNK
    tail = _N - nfull * _CHUNK
    ro_scope = jax.named_scope("sc_readout")
    ro_scope.__enter__()
    for k in range(-(-(nfull + 1) // _NS)):
        blk = s + k * _NS
        r0 = pl.multiple_of(blk * _CHUNK, _CHUNK)
        o0 = pl.multiple_of(c * _N + r0, 16)

        @pl.when(blk < nfull)
        def _():
            pltpu.sync_copy(agg_sh.at[pl.ds(r0, _CHUNK)], rows_a)
            pltpu.sync_copy(rows_a, out_hbm.at[pl.ds(o0, _CHUNK)])

        @pl.when(blk == nfull)
        def _():
            pltpu.sync_copy(agg_sh.at[pl.ds(r0, tail)], rows_a.at[pl.ds(0, tail)])
            pltpu.sync_copy(rows_a.at[pl.ds(0, tail)],
                            out_hbm.at[pl.ds(o0, tail)])

    ro_scope.__exit__(None, None, None)


@functools.cache
def _sc_segment_sum():
    mesh = plsc.VectorSubcoreMesh(
        core_axis_name="c", subcore_axis_name="s", num_cores=_NC, num_subcores=_NS
    )
    return pl.kernel(
        _sc_body,
        out_type=jax.ShapeDtypeStruct((_NC * _N, _D), jnp.float32),
        mesh=mesh,
        scratch_types=[
            pltpu.VMEM((_CHUNKS_PER_TILE, _CHUNK), jnp.int32),
            pltpu.VMEM((1, _CHUNK), jnp.int32),
            pltpu.VMEM((1, _CHUNK), jnp.int32),
            pltpu.VMEM((1, _CHUNK), jnp.int32),
            pltpu.VMEM((1, _CHUNK), jnp.int32),
            pltpu.VMEM((_CHUNK, _D), jnp.float32),
            pltpu.VMEM((_CHUNK, _D), jnp.float32),
            pltpu.VMEM_SHARED((_AGG_ROWS, _D), jnp.float32),
            pltpu.SemaphoreType.DMA,
            pltpu.SemaphoreType.DMA,
            pltpu.SemaphoreType.DMA,
        ],
    )


_TC_ROWS = 1000


def _tc_body(x_ref, agg_ref, wm_ref, ws_ref, b_ref, o_ref):
    agg = agg_ref[0] + agg_ref[1]
    acc = jnp.dot(x_ref[...], ws_ref[...], preferred_element_type=jnp.float32)
    acc = acc + jnp.dot(agg, wm_ref[...], preferred_element_type=jnp.float32)
    o_ref[...] = jnp.maximum(acc + b_ref[...], 0.0)


@jax.jit
def _tc_combine(x, agg2, W_msg, W_self, b2):
    return pl.pallas_call(
        _tc_body,
        grid=(_N // _TC_ROWS,),
        in_specs=[
            pl.BlockSpec((_TC_ROWS, _D), lambda i: (i, 0)),
            pl.BlockSpec((_NC, _TC_ROWS, _D), lambda i: (0, i, 0)),
            pl.BlockSpec((_D, _D), lambda i: (0, 0)),
            pl.BlockSpec((_D, _D), lambda i: (0, 0)),
            pl.BlockSpec((1, _D), lambda i: (0, 0)),
        ],
        out_specs=pl.BlockSpec((_TC_ROWS, _D), lambda i: (i, 0)),
        out_shape=jax.ShapeDtypeStruct((_N, _D), jnp.float32),
    )(x, agg2, W_msg, W_self, b2)


def kernel(x, edge_index, W_msg, W_self, b):
    src = edge_index[0].astype(jnp.int32)
    dst = edge_index[1].astype(jnp.int32)
    packed = jnp.concatenate([
        (src << 16) | dst,
        jnp.full((_E_PAD - _E,), _N, jnp.int32),
    ]).reshape(_NW, _CHUNKS_PER_TILE, _CHUNK)
    zeros_blk = jnp.zeros((_CHUNK, _D), jnp.float32)
    agg2 = _sc_segment_sum()(x, packed, zeros_blk).reshape(_NC, _N, _D)
    return _tc_combine(x, agg2, W_msg, W_self, b.reshape(1, _D))


# speedup vs baseline: 2.7982x; 2.7951x over previous
import functools

import jax
import jax.numpy as jnp
from jax import lax
from jax.experimental import pallas as pl
from jax.experimental.pallas import tpu as pltpu
from jax.experimental.pallas import tpu_sc as plsc

_N = 10000
_D = 128
_E = 320000
_NC = 2
_NS = 16
_NW = _NC * _NS
_L = 16
_CHUNK = 128
_CHUNKS_PER_TILE = -(-_E // (_NW * _CHUNK))
_CHUNKS_PER_TILE += _CHUNKS_PER_TILE % 2
_EDGES_PER_TILE = _CHUNKS_PER_TILE * _CHUNK
_E_PAD = _NW * _EDGES_PER_TILE
_AGG_ROWS = 10112
_ZBLOCKS = _AGG_ROWS // _CHUNK


def _unpack_chunk(pack_v, j, sidx, didx):
    for l in range(_CHUNK // _L):
        w = pack_v[j, pl.ds(l * _L, _L)]
        sidx[0, pl.ds(l * _L, _L)] = lax.shift_right_logical(w, 16)
        didx[0, pl.ds(l * _L, _L)] = lax.bitwise_and(w, 0xFFFF)


def _sc_body(x_hbm, pack_hbm, zeros_hbm, out_hbm,
             pack_v, sidx_a, sidx_b, didx_a, didx_b, rows_a, rows_b,
             agg_sh, sem_ga, sem_gb, sem_s):
    c = lax.axis_index("c")
    s = lax.axis_index("s")
    wid = c * _NS + s
    n = _CHUNKS_PER_TILE

    with jax.named_scope("sc_zero"):
        pltpu.sync_copy(zeros_hbm, rows_a)
        for k in range(-(-_ZBLOCKS // _NS)):
            blk = s + k * _NS

            @pl.when(blk < _ZBLOCKS)
            def _():
                pltpu.sync_copy(rows_a, agg_sh.at[pl.ds(blk * _CHUNK, _CHUNK)])

        plsc.subcore_barrier()

    pltpu.sync_copy(pack_hbm.at[wid], pack_v)

    bufs = ((rows_a, sidx_a, didx_a, sem_ga), (rows_b, sidx_b, didx_b, sem_gb))

    edge_scope = jax.named_scope("sc_edges")
    edge_scope.__enter__()
    _unpack_chunk(pack_v, 0, sidx_a, didx_a)
    _unpack_chunk(pack_v, 1, sidx_b, didx_b)
    pltpu.async_copy(x_hbm.at[sidx_a.at[0]], rows_a, sem_ga)
    pltpu.async_copy(x_hbm.at[sidx_b.at[0]], rows_b, sem_gb)

    def pair_body(i, carry):
        j0 = 2 * i
        for bi, (rows, sidx, didx, sem_g) in enumerate(bufs):
            j = j0 + bi
            pltpu.make_async_copy(x_hbm.at[sidx.at[0]], rows, sem_g).wait()
            pltpu.async_copy(rows, agg_sh.at[didx.at[0]], sem_s, add=True).wait()

            @pl.when(j + 2 < n)
            def _():
                _unpack_chunk(pack_v, j + 2, sidx, didx)
                pltpu.async_copy(x_hbm.at[sidx.at[0]], rows, sem_g)

        return carry

    lax.fori_loop(0, n // 2, pair_body, 0, unroll=False)
    edge_scope.__exit__(None, None, None)

    plsc.subcore_barrier()

    nfull = _N // _CHUNK
    tail = _N - nfull * _CHUNK
    ro_scope = jax.named_scope("sc_readout")
    ro_scope.__enter__()
    for k in range(-(-(nfull + 1) // _NS)):
        blk = s + k * _NS
        r0 = pl.multiple_of(blk * _CHUNK, _CHUNK)
        o0 = pl.multiple_of(c * _N + r0, 16)

        @pl.when(blk < nfull)
        def _():
            pltpu.sync_copy(agg_sh.at[pl.ds(r0, _CHUNK)], rows_a)
            pltpu.sync_copy(rows_a, out_hbm.at[pl.ds(o0, _CHUNK)])

        @pl.when(blk == nfull)
        def _():
            pltpu.sync_copy(agg_sh.at[pl.ds(r0, tail)], rows_a.at[pl.ds(0, tail)])
            pltpu.sync_copy(rows_a.at[pl.ds(0, tail)],
                            out_hbm.at[pl.ds(o0, tail)])

    ro_scope.__exit__(None, None, None)


@functools.cache
def _sc_segment_sum():
    mesh = plsc.VectorSubcoreMesh(
        core_axis_name="c", subcore_axis_name="s", num_cores=_NC, num_subcores=_NS
    )
    return pl.kernel(
        _sc_body,
        out_type=jax.ShapeDtypeStruct((_NC * _N, _D), jnp.float32),
        mesh=mesh,
        scratch_types=[
            pltpu.VMEM((_CHUNKS_PER_TILE, _CHUNK), jnp.int32),
            pltpu.VMEM((1, _CHUNK), jnp.int32),
            pltpu.VMEM((1, _CHUNK), jnp.int32),
            pltpu.VMEM((1, _CHUNK), jnp.int32),
            pltpu.VMEM((1, _CHUNK), jnp.int32),
            pltpu.VMEM((_CHUNK, _D), jnp.float32),
            pltpu.VMEM((_CHUNK, _D), jnp.float32),
            pltpu.VMEM_SHARED((_AGG_ROWS, _D), jnp.float32),
            pltpu.SemaphoreType.DMA,
            pltpu.SemaphoreType.DMA,
            pltpu.SemaphoreType.DMA,
        ],
    )


_TC_ROWS = 1000


def _tc_body(x_ref, agg_ref, wm_ref, ws_ref, b_ref, o_ref):
    agg = agg_ref[0] + agg_ref[1]
    acc = jnp.dot(x_ref[...], ws_ref[...], preferred_element_type=jnp.float32)
    acc = acc + jnp.dot(agg, wm_ref[...], preferred_element_type=jnp.float32)
    o_ref[...] = jnp.maximum(acc + b_ref[...], 0.0)


@jax.jit
def _tc_combine(x, agg2, W_msg, W_self, b2):
    return pl.pallas_call(
        _tc_body,
        grid=(_N // _TC_ROWS,),
        in_specs=[
            pl.BlockSpec((_TC_ROWS, _D), lambda i: (i, 0)),
            pl.BlockSpec((_NC, _TC_ROWS, _D), lambda i: (0, i, 0)),
            pl.BlockSpec((_D, _D), lambda i: (0, 0)),
            pl.BlockSpec((_D, _D), lambda i: (0, 0)),
            pl.BlockSpec((1, _D), lambda i: (0, 0)),
        ],
        out_specs=pl.BlockSpec((_TC_ROWS, _D), lambda i: (i, 0)),
        out_shape=jax.ShapeDtypeStruct((_N, _D), jnp.float32),
    )(x, agg2, W_msg, W_self, b2)


def kernel(x, edge_index, W_msg, W_self, b):
    src = edge_index[0].astype(jnp.int32)
    dst = edge_index[1].astype(jnp.int32)
    pad_idx = jnp.arange(_E_PAD - _E, dtype=jnp.int32)
    pad_packed = ((pad_idx & 8191) << 16) | (_N + pad_idx % (_AGG_ROWS - _N))
    packed = jnp.concatenate([
        (src << 16) | dst,
        pad_packed,
    ]).reshape(_NW, _CHUNKS_PER_TILE, _CHUNK)
    zeros_blk = jnp.zeros((_CHUNK, _D), jnp.float32)
    agg2 = _sc_segment_sum()(x, packed, zeros_blk).reshape(_NC, _N, _D)
    return _tc_combine(x, agg2, W_msg, W_self, b.reshape(1, _D))
